# baseline (speedup 1.0000x reference)
import functools

import jax
import jax.numpy as jnp
from jax import lax
from jax.experimental import pallas as pl
from jax.experimental.pallas import tpu as pltpu
from jax.experimental.pallas import tpu_sc as plsc

N = 100000
NUM_Z = 119
NC = 2
NS = 16
L = 16
NW = NC * NS
CHUNK = 3136
HALF = CHUNK // 2
assert NW * CHUNK >= N and HALF % L == 0 and (N - CHUNK) % 8 == 0


def _sc_scale_shift(out_hbm, z_hbm, scale_hbm, shift_hbm, res_hbm,
                    out_v, z_v, scale_v, shift_v, sem_a, sem_b, sem_o):
    wid = lax.axis_index("s") * NC + lax.axis_index("c")
    base = jnp.minimum(wid * CHUNK, N - CHUNK)

    cp_z1 = pltpu.make_async_copy(
        z_hbm.at[pl.ds(base, HALF)], z_v.at[pl.ds(0, HALF)], sem_a)
    cp_o1 = pltpu.make_async_copy(
        out_hbm.at[pl.ds(base, HALF)], out_v.at[pl.ds(0, HALF)], sem_a)
    cp_s = pltpu.make_async_copy(scale_hbm, scale_v, sem_a)
    cp_t = pltpu.make_async_copy(shift_hbm, shift_v, sem_a)
    cp_z2 = pltpu.make_async_copy(
        z_hbm.at[pl.ds(base + HALF, HALF)], z_v.at[pl.ds(HALF, HALF)], sem_b)
    cp_o2 = pltpu.make_async_copy(
        out_hbm.at[pl.ds(base + HALF, HALF)], out_v.at[pl.ds(HALF, HALF)], sem_b)
    cp_z1.start()
    cp_o1.start()
    cp_s.start()
    cp_t.start()
    cp_z2.start()
    cp_o2.start()

    cp_z1.wait()
    cp_o1.wait()
    cp_s.wait()
    cp_t.wait()

    @plsc.parallel_loop(0, HALF, step=L, unroll=8)
    def body1(off):
        idx = z_v[pl.ds(off, L)]
        s = plsc.load_gather(scale_v, [idx])
        t = plsc.load_gather(shift_v, [idx])
        out_v[pl.ds(off, L)] = out_v[pl.ds(off, L)] * s + t

    cp_r1 = pltpu.make_async_copy(
        out_v.at[pl.ds(0, HALF)], res_hbm.at[pl.ds(base, HALF)], sem_o)
    cp_r1.start()

    cp_z2.wait()
    cp_o2.wait()

    @plsc.parallel_loop(HALF, CHUNK, step=L, unroll=8)
    def body2(off):
        idx = z_v[pl.ds(off, L)]
        s = plsc.load_gather(scale_v, [idx])
        t = plsc.load_gather(shift_v, [idx])
        out_v[pl.ds(off, L)] = out_v[pl.ds(off, L)] * s + t

    cp_r2 = pltpu.make_async_copy(
        out_v.at[pl.ds(HALF, HALF)], res_hbm.at[pl.ds(base + HALF, HALF)], sem_o)
    cp_r2.start()
    cp_r1.wait()
    cp_r2.wait()


@jax.jit
def _run(output_flat, z_i32, scale_flat, shift_flat):
    k = pl.kernel(
        _sc_scale_shift,
        out_type=jax.ShapeDtypeStruct((N,), jnp.float32),
        mesh=plsc.VectorSubcoreMesh(core_axis_name="c", subcore_axis_name="s"),
        compiler_params=pltpu.CompilerParams(
            needs_layout_passes=False,
            disable_bounds_checks=True,
            disable_semaphore_checks=True,
            skip_device_barrier=True,
        ),
        scratch_types=[
            pltpu.VMEM((CHUNK,), jnp.float32),
            pltpu.VMEM((CHUNK,), jnp.int32),
            pltpu.VMEM((NUM_Z,), jnp.float32),
            pltpu.VMEM((NUM_Z,), jnp.float32),
            pltpu.SemaphoreType.DMA,
            pltpu.SemaphoreType.DMA,
            pltpu.SemaphoreType.DMA,
        ],
    )
    return k(output_flat, z_i32, scale_flat, shift_flat)


def kernel(output, z, scale_w, shift_w):
    res = _run(
        output.reshape(N),
        z.astype(jnp.int32),
        scale_w.reshape(NUM_Z),
        shift_w.reshape(NUM_Z),
    )
    return res.reshape(N, 1)


# speedup vs baseline: 1.0009x; 1.0009x over previous
import jax
import jax.numpy as jnp
from jax import lax
from jax.experimental import pallas as pl
from jax.experimental.pallas import tpu as pltpu
from jax.experimental.pallas import tpu_sc as plsc

N = 100000
NUM_Z = 119
NC = 2
NS = 16
L = 16
NW = NC * NS
CHUNK = 3136
assert NW * CHUNK >= N and CHUNK % L == 0 and (N - CHUNK) % 8 == 0


def _sc_scale_shift(out_hbm, z_hbm, scale_hbm, shift_hbm, res_hbm,
                    out_v, z_v, scale_v, shift_v, sem):
    wid = lax.axis_index("s") * NC + lax.axis_index("c")
    base = jnp.minimum(wid * CHUNK, N - CHUNK)

    cp_z = pltpu.make_async_copy(z_hbm.at[pl.ds(base, CHUNK)], z_v, sem)
    cp_o = pltpu.make_async_copy(out_hbm.at[pl.ds(base, CHUNK)], out_v, sem)
    cp_s = pltpu.make_async_copy(scale_hbm, scale_v, sem)
    cp_t = pltpu.make_async_copy(shift_hbm, shift_v, sem)
    cp_z.start()
    cp_o.start()
    cp_s.start()
    cp_t.start()
    cp_z.wait()
    cp_o.wait()
    cp_s.wait()
    cp_t.wait()

    @plsc.parallel_loop(0, CHUNK, step=L, unroll=8)
    def body(off):
        idx = z_v[pl.ds(off, L)]
        s = plsc.load_gather(scale_v, [idx])
        t = plsc.load_gather(shift_v, [idx])
        out_v[pl.ds(off, L)] = out_v[pl.ds(off, L)] * s + t

    pltpu.sync_copy(out_v, res_hbm.at[pl.ds(base, CHUNK)])


@jax.jit
def _run(output_flat, z_i32, scale_flat, shift_flat):
    k = pl.kernel(
        _sc_scale_shift,
        out_type=jax.ShapeDtypeStruct((N,), jnp.float32),
        mesh=plsc.VectorSubcoreMesh(core_axis_name="c", subcore_axis_name="s"),
        compiler_params=pltpu.CompilerParams(
            needs_layout_passes=False,
            disable_bounds_checks=True,
            disable_semaphore_checks=True,
            skip_device_barrier=True,
        ),
        scratch_types=[
            pltpu.VMEM((CHUNK,), jnp.float32),
            pltpu.VMEM((CHUNK,), jnp.int32),
            pltpu.VMEM((NUM_Z,), jnp.float32),
            pltpu.VMEM((NUM_Z,), jnp.float32),
            pltpu.SemaphoreType.DMA,
        ],
    )
    return k(output_flat, z_i32, scale_flat, shift_flat)


def kernel(output, z, scale_w, shift_w):
    res = _run(
        output.reshape(N),
        z.astype(jnp.int32),
        scale_w.reshape(NUM_Z),
        shift_w.reshape(NUM_Z),
    )
    return res.reshape(N, 1)
